# baseline (speedup 1.0000x reference)
import jax
import jax.numpy as jnp
from jax.experimental import pallas as pl

_B, _N, _F_ATOM, _F_PHYS = 1024, 100, 38, 3
_F_TOT = _F_ATOM + _F_PHYS
_ROW = _N * _F_TOT


def _fused_kernel(x_ref, P_ref, br_ref, Wc_ref, bc_ref, W1_ref, b1_ref,
                  W5_ref, b5_ref, W6_ref, b6_ref, W7h_ref, W7p_ref, b7_ref,
                  out_ref):
    x = x_ref[...]
    h = jnp.dot(x, P_ref[...], preferred_element_type=jnp.float32)
    h = jax.nn.relu(h + br_ref[...])
    g = jax.nn.relu(jnp.dot(h, Wc_ref[...], preferred_element_type=jnp.float32)
                    + bc_ref[...]) * float(_N)
    d1 = jax.nn.relu(jnp.dot(g, W1_ref[...], preferred_element_type=jnp.float32)
                     + b1_ref[...])
    d5 = jnp.dot(d1, W5_ref[...], preferred_element_type=jnp.float32) + b5_ref[...]
    mv = jnp.dot(d5, W6_ref[...], preferred_element_type=jnp.float32) + b6_ref[...]
    phys = x[:, _F_ATOM:_F_ATOM + _F_PHYS]
    out = mv * W7h_ref[0, 0] + jnp.dot(phys, W7p_ref[...],
                                       preferred_element_type=jnp.float32)
    out_ref[...] = out + b7_ref[...]


def kernel(inputs, W_rule, b_rule, W_conv, b_conv, W1, b1, W5, b5, W6, b6,
           W7, b7):
    B, N, F_tot = inputs.shape
    R = W_rule.shape[1]
    x2d = inputs.reshape(B, N * F_tot)
    P = jnp.concatenate(
        [W_rule / float(_N),
         jnp.zeros((_F_PHYS, R), dtype=W_rule.dtype)], axis=0)
    P = jnp.tile(P, (N, 1))

    bB = 128
    grid = (B // bB,)
    out = pl.pallas_call(
        _fused_kernel,
        grid=grid,
        in_specs=[
            pl.BlockSpec((bB, N * F_tot), lambda i: (i, 0)),
            pl.BlockSpec((N * F_tot, R), lambda i: (0, 0)),
            pl.BlockSpec((1, R), lambda i: (0, 0)),
            pl.BlockSpec(W_conv.shape, lambda i: (0, 0)),
            pl.BlockSpec((1, W_conv.shape[1]), lambda i: (0, 0)),
            pl.BlockSpec(W1.shape, lambda i: (0, 0)),
            pl.BlockSpec((1, W1.shape[1]), lambda i: (0, 0)),
            pl.BlockSpec(W5.shape, lambda i: (0, 0)),
            pl.BlockSpec((1, W5.shape[1]), lambda i: (0, 0)),
            pl.BlockSpec(W6.shape, lambda i: (0, 0)),
            pl.BlockSpec((1, 1), lambda i: (0, 0)),
            pl.BlockSpec((1, 1), lambda i: (0, 0)),
            pl.BlockSpec((_F_PHYS, 1), lambda i: (0, 0)),
            pl.BlockSpec((1, 1), lambda i: (0, 0)),
        ],
        out_specs=pl.BlockSpec((bB, 1), lambda i: (i, 0)),
        out_shape=jax.ShapeDtypeStruct((B, 1), jnp.float32),
    )(x2d, P, b_rule.reshape(1, -1), W_conv, b_conv.reshape(1, -1),
      W1, b1.reshape(1, -1), W5, b5.reshape(1, -1), W6, b6.reshape(1, -1),
      W7[0:1, :], W7[1:4, :], b7.reshape(1, -1))
    return out


# speedup vs baseline: 1.2453x; 1.2453x over previous
import jax
import jax.numpy as jnp
from jax.experimental import pallas as pl

_B, _N, _F_ATOM, _F_PHYS = 1024, 100, 38, 3
_F_TOT = _F_ATOM + _F_PHYS


def _fused_kernel(x_ref, Wr_ref, br_ref, Wc_ref, bc_ref, W1_ref, b1_ref,
                  W5_ref, b5_ref, W6_ref, b6_ref, W7h_ref, W7p_ref, b7_ref,
                  out_ref):
    x = x_ref[...]
    xbar = jnp.sum(x, axis=1) * (1.0 / _N)
    xb = xbar[:, :_F_ATOM]
    phys = x[:, 0, _F_ATOM:]
    h = jax.nn.relu(jnp.dot(xb, Wr_ref[...], preferred_element_type=jnp.float32)
                    + br_ref[...])
    g = jax.nn.relu(jnp.dot(h, Wc_ref[...], preferred_element_type=jnp.float32)
                    + bc_ref[...]) * float(_N)
    d1 = jax.nn.relu(jnp.dot(g, W1_ref[...], preferred_element_type=jnp.float32)
                     + b1_ref[...])
    d5 = jnp.dot(d1, W5_ref[...], preferred_element_type=jnp.float32) + b5_ref[...]
    mv = jnp.dot(d5, W6_ref[...], preferred_element_type=jnp.float32) + b6_ref[...]
    out = mv * W7h_ref[0, 0] + jnp.dot(phys, W7p_ref[...],
                                       preferred_element_type=jnp.float32)
    out_ref[...] = out + b7_ref[...]


def kernel(inputs, W_rule, b_rule, W_conv, b_conv, W1, b1, W5, b5, W6, b6,
           W7, b7):
    B, N, F_tot = inputs.shape
    R = W_rule.shape[1]

    bB = 128
    grid = (B // bB,)
    out = pl.pallas_call(
        _fused_kernel,
        grid=grid,
        in_specs=[
            pl.BlockSpec((bB, N, F_tot), lambda i: (i, 0, 0)),
            pl.BlockSpec(W_rule.shape, lambda i: (0, 0)),
            pl.BlockSpec((1, R), lambda i: (0, 0)),
            pl.BlockSpec(W_conv.shape, lambda i: (0, 0)),
            pl.BlockSpec((1, W_conv.shape[1]), lambda i: (0, 0)),
            pl.BlockSpec(W1.shape, lambda i: (0, 0)),
            pl.BlockSpec((1, W1.shape[1]), lambda i: (0, 0)),
            pl.BlockSpec(W5.shape, lambda i: (0, 0)),
            pl.BlockSpec((1, W5.shape[1]), lambda i: (0, 0)),
            pl.BlockSpec(W6.shape, lambda i: (0, 0)),
            pl.BlockSpec((1, 1), lambda i: (0, 0)),
            pl.BlockSpec((1, 1), lambda i: (0, 0)),
            pl.BlockSpec((_F_PHYS, 1), lambda i: (0, 0)),
            pl.BlockSpec((1, 1), lambda i: (0, 0)),
        ],
        out_specs=pl.BlockSpec((bB, 1), lambda i: (i, 0)),
        out_shape=jax.ShapeDtypeStruct((B, 1), jnp.float32),
    )(inputs, W_rule, b_rule.reshape(1, -1), W_conv, b_conv.reshape(1, -1),
      W1, b1.reshape(1, -1), W5, b5.reshape(1, -1), W6, b6.reshape(1, -1),
      W7[0:1, :], W7[1:4, :], b7.reshape(1, -1))
    return out


# speedup vs baseline: 1.2511x; 1.0047x over previous
import jax
import jax.numpy as jnp
from jax.experimental import pallas as pl

_B, _N, _F_ATOM, _F_PHYS = 1024, 100, 38, 3
_F_TOT = _F_ATOM + _F_PHYS


def _fused_kernel(x_ref, Wr_ref, br_ref, Wc_ref, bc_ref, W1_ref, b1_ref,
                  W5_ref, b5_ref, W6_ref, b6_ref, W7h_ref, W7p_ref, b7_ref,
                  out_ref):
    x = x_ref[...]
    xbar = jnp.sum(x, axis=1) * (1.0 / _N)
    xb = xbar[:, :_F_ATOM]
    phys = x[:, 0, _F_ATOM:]
    h = jax.nn.relu(jnp.dot(xb, Wr_ref[...], preferred_element_type=jnp.float32)
                    + br_ref[...])
    g = jax.nn.relu(jnp.dot(h, Wc_ref[...], preferred_element_type=jnp.float32)
                    + bc_ref[...]) * float(_N)
    d1 = jax.nn.relu(jnp.dot(g, W1_ref[...], preferred_element_type=jnp.float32)
                     + b1_ref[...])
    d5 = jnp.dot(d1, W5_ref[...], preferred_element_type=jnp.float32) + b5_ref[...]
    mv = jnp.dot(d5, W6_ref[...], preferred_element_type=jnp.float32) + b6_ref[...]
    out = mv * W7h_ref[0, 0] + jnp.dot(phys, W7p_ref[...],
                                       preferred_element_type=jnp.float32)
    out_ref[...] = out + b7_ref[...]


def kernel(inputs, W_rule, b_rule, W_conv, b_conv, W1, b1, W5, b5, W6, b6,
           W7, b7):
    B, N, F_tot = inputs.shape
    R = W_rule.shape[1]

    bB = 256
    grid = (B // bB,)
    out = pl.pallas_call(
        _fused_kernel,
        grid=grid,
        in_specs=[
            pl.BlockSpec((bB, N, F_tot), lambda i: (i, 0, 0)),
            pl.BlockSpec(W_rule.shape, lambda i: (0, 0)),
            pl.BlockSpec((1, R), lambda i: (0, 0)),
            pl.BlockSpec(W_conv.shape, lambda i: (0, 0)),
            pl.BlockSpec((1, W_conv.shape[1]), lambda i: (0, 0)),
            pl.BlockSpec(W1.shape, lambda i: (0, 0)),
            pl.BlockSpec((1, W1.shape[1]), lambda i: (0, 0)),
            pl.BlockSpec(W5.shape, lambda i: (0, 0)),
            pl.BlockSpec((1, W5.shape[1]), lambda i: (0, 0)),
            pl.BlockSpec(W6.shape, lambda i: (0, 0)),
            pl.BlockSpec((1, 1), lambda i: (0, 0)),
            pl.BlockSpec((1, 1), lambda i: (0, 0)),
            pl.BlockSpec((_F_PHYS, 1), lambda i: (0, 0)),
            pl.BlockSpec((1, 1), lambda i: (0, 0)),
        ],
        out_specs=pl.BlockSpec((bB, 1), lambda i: (i, 0)),
        out_shape=jax.ShapeDtypeStruct((B, 1), jnp.float32),
    )(inputs, W_rule, b_rule.reshape(1, -1), W_conv, b_conv.reshape(1, -1),
      W1, b1.reshape(1, -1), W5, b5.reshape(1, -1), W6, b6.reshape(1, -1),
      W7[0:1, :], W7[1:4, :], b7.reshape(1, -1))
    return out
